# Initial kernel scaffold; baseline (speedup 1.0000x reference)
import jax
import jax.numpy as jnp
from jax.experimental import pallas as pl


def kernel(X, edges, W_heads, a_heads, W_out, a_out):
    raise NotImplementedError("write your pallas kernel here")


# speedup vs baseline: 1.9672x; 1.9672x over previous
import functools

import jax
import jax.numpy as jnp
from jax import lax
from jax.experimental import pallas as pl
from jax.experimental.pallas import tpu as pltpu
from jax.experimental.pallas import tpu_sc as plsc

N_NODES = 10000
N_EDGES = 160000
INPUT_DIM = 256
HIDDEN_DIM = 64
OUTPUT_DIM = 64
NUM_HEADS = 8
ALPHA = 0.2

NW = 16
EPW = N_EDGES // NW
CH = 80
NCH = EPW // CH
NPH = 2560
TRASH = NPH
NACC = NPH + 128
RPW = NPH // NW
RCH = 160
NRCH = RPW // RCH
ZPW = NACC // NW
AW = 128


def _elu(x):
    return jnp.where(x > 0, x, jnp.exp(x) - 1.0)


def _l1_body(x_ref, w_ref, a_ref, h_ref, as_ref, ad_ref):
    h = jnp.dot(x_ref[...], w_ref[0], preferred_element_type=jnp.float32)
    h_ref[0] = h
    a = a_ref[0]
    as_ref[0] = jnp.dot(h, a[:HIDDEN_DIM], preferred_element_type=jnp.float32)
    ad_ref[0] = jnp.dot(h, a[HIDDEN_DIM:], preferred_element_type=jnp.float32)


def _layer1(X, W_heads, a_heads):
    blk = 1000
    nb = N_NODES // blk
    return pl.pallas_call(
        _l1_body,
        grid=(NUM_HEADS, nb),
        in_specs=[
            pl.BlockSpec((blk, INPUT_DIM), lambda h, n: (n, 0)),
            pl.BlockSpec((1, INPUT_DIM, HIDDEN_DIM), lambda h, n: (h, 0, 0)),
            pl.BlockSpec((1, 2 * HIDDEN_DIM, 1), lambda h, n: (h, 0, 0)),
        ],
        out_specs=[
            pl.BlockSpec((1, blk, HIDDEN_DIM), lambda h, n: (h, n, 0)),
            pl.BlockSpec((1, blk, 1), lambda h, n: (h, n, 0)),
            pl.BlockSpec((1, blk, 1), lambda h, n: (h, n, 0)),
        ],
        out_shape=[
            jax.ShapeDtypeStruct((NUM_HEADS, N_NODES, HIDDEN_DIM), jnp.float32),
            jax.ShapeDtypeStruct((NUM_HEADS, N_NODES, 1), jnp.float32),
            jax.ShapeDtypeStruct((NUM_HEADS, N_NODES, 1), jnp.float32),
        ],
    )(X, W_heads, a_heads)


def _l2_body(g_ref, w_ref, a_ref, h_ref, as_ref, ad_ref):
    acc = jnp.zeros((g_ref.shape[1], OUTPUT_DIM), dtype=jnp.float32)
    for hh in range(NUM_HEADS):
        g = _elu(_elu(g_ref[hh]))
        acc = acc + jnp.dot(
            g, w_ref[hh * HIDDEN_DIM:(hh + 1) * HIDDEN_DIM],
            preferred_element_type=jnp.float32)
    h_ref[...] = acc
    a = a_ref[...]
    as_ref[...] = jnp.dot(acc, a[:OUTPUT_DIM], preferred_element_type=jnp.float32)
    ad_ref[...] = jnp.dot(acc, a[OUTPUT_DIM:], preferred_element_type=jnp.float32)


def _layer2(agg, W_out, a_out):
    blk = 1000
    nb = N_NODES // blk
    return pl.pallas_call(
        _l2_body,
        grid=(nb,),
        in_specs=[
            pl.BlockSpec((NUM_HEADS, blk, HIDDEN_DIM), lambda n: (0, n, 0)),
            pl.BlockSpec((NUM_HEADS * HIDDEN_DIM, OUTPUT_DIM), lambda n: (0, 0)),
            pl.BlockSpec((2 * OUTPUT_DIM, 1), lambda n: (0, 0)),
        ],
        out_specs=[
            pl.BlockSpec((blk, OUTPUT_DIM), lambda n: (n, 0)),
            pl.BlockSpec((blk, 1), lambda n: (n, 0)),
            pl.BlockSpec((blk, 1), lambda n: (n, 0)),
        ],
        out_shape=[
            jax.ShapeDtypeStruct((N_NODES, OUTPUT_DIM), jnp.float32),
            jax.ShapeDtypeStruct((N_NODES, 1), jnp.float32),
            jax.ShapeDtypeStruct((N_NODES, 1), jnp.float32),
        ],
    )(agg, W_out, a_out)


def _sc_body(base, h_hbm, src_hbm, dst_hbm, out_hbm,
             src2d, dst2d, dsta_v, rows_v, adr_v, w_v, fin_v, o_v,
             acc_sh, sem):
    wid = lax.axis_index("s") + lax.axis_index("c")

    pltpu.sync_copy(src_hbm.at[wid], src2d)
    pltpu.sync_copy(dst_hbm.at[wid], dst2d)

    def _zrow(r, c):
        for cc in range(AW // 16):
            fin_v[r, pl.ds(cc * 16, 16)] = jnp.zeros((16,), jnp.float32)
        return c
    lax.fori_loop(0, RCH, _zrow, 0)
    z0 = wid * ZPW
    pltpu.sync_copy(fin_v, acc_sh.at[pl.ds(z0, RCH)])
    pltpu.sync_copy(fin_v.at[pl.ds(0, 8)], acc_sh.at[pl.ds(z0 + RCH, 8)])
    plsc.subcore_barrier()

    lane = lax.iota(jnp.int32, 16)
    zeros16 = jnp.zeros((16,), jnp.float32)

    def _chunk(c, carry):
        pltpu.async_copy(h_hbm.at[src2d.at[c]], rows_v, sem).wait()
        pltpu.async_copy(h_hbm.at[dst2d.at[c]], adr_v, sem).wait()

        for k in range(CH // 16):
            d16 = dst2d[c, pl.ds(k * 16, 16)]
            adj = d16 - base
            ok = (adj >= 0) & (adj < NPH)
            dsta_v[0, pl.ds(k * 16, 16)] = jnp.where(ok, adj, TRASH)

        def _edge(i, cc2):
            a = rows_v[i, pl.ds(HIDDEN_DIM, 16)]
            b = adr_v[i, pl.ds(HIDDEN_DIM, 16)]
            x = jnp.full((16,), a[0] + b[1], jnp.float32)
            e = jnp.where(x >= 0, x, ALPHA * x)
            ev = jnp.exp(e)
            for cc in range(HIDDEN_DIM // 16):
                w_v[i, pl.ds(cc * 16, 16)] = (
                    ev * rows_v[i, pl.ds(cc * 16, 16)])
            w_v[i, pl.ds(HIDDEN_DIM, 16)] = jnp.where(lane == 0, ev, zeros16)
            return cc2
        lax.fori_loop(0, CH, _edge, 0)

        pltpu.sync_copy(w_v, acc_sh.at[dsta_v.at[0]], add=True)
        return carry
    lax.fori_loop(0, NCH, _chunk, 0)
    plsc.subcore_barrier()

    row0 = wid * RPW
    for j in range(NRCH):
        pltpu.sync_copy(acc_sh.at[pl.ds(row0 + j * RCH, RCH)], fin_v)

        def _frow(r, c):
            s = fin_v[r, pl.ds(HIDDEN_DIM, 16)][0]
            den = jnp.full((16,), s, jnp.float32) + 1e-16
            for cc in range(HIDDEN_DIM // 16):
                o_v[r, pl.ds(cc * 16, 16)] = fin_v[r, pl.ds(cc * 16, 16)] / den
            return c
        lax.fori_loop(0, RCH, _frow, 0)
        pltpu.sync_copy(o_v, out_hbm.at[pl.ds(row0 + j * RCH, RCH)])


def _make_sc_gat(base):
    mesh = plsc.VectorSubcoreMesh(core_axis_name="c", subcore_axis_name="s",
                                  num_cores=1)
    return functools.partial(
        pl.kernel,
        out_type=jax.ShapeDtypeStruct((NPH, HIDDEN_DIM), jnp.float32),
        mesh=mesh,
        scratch_types=[
            pltpu.VMEM((NCH, CH), jnp.int32),
            pltpu.VMEM((NCH, CH), jnp.int32),
            pltpu.VMEM((8, CH), jnp.int32),
            pltpu.VMEM((CH, AW), jnp.float32),
            pltpu.VMEM((CH, AW), jnp.float32),
            pltpu.VMEM((CH, AW), jnp.float32),
            pltpu.VMEM((RCH, AW), jnp.float32),
            pltpu.VMEM((RCH, HIDDEN_DIM), jnp.float32),
            pltpu.VMEM_SHARED((NACC, AW), jnp.float32),
            pltpu.SemaphoreType.DMA,
        ],
    )(functools.partial(_sc_body, base))


_sc_gat_passes = [_make_sc_gat(p * NPH) for p in range(4)]


def _sc_gat(h_ext, src3, dst3):
    parts = [f(h_ext, src3, dst3) for f in _sc_gat_passes]
    return jnp.concatenate(parts, axis=0)[:N_NODES]


def kernel(X, edges, W_heads, a_heads, W_out, a_out):
    src3 = edges[0].reshape(NW, NCH, CH)
    dst3 = edges[1].reshape(NW, NCH, CH)

    def _table(h, a_s, a_d):
        return jnp.concatenate(
            [h, a_s, a_d,
             jnp.zeros((N_NODES, AW - HIDDEN_DIM - 2), jnp.float32)],
            axis=1)

    h_all, as_all, ad_all = _layer1(X, W_heads, a_heads)
    aggs = []
    for i in range(NUM_HEADS):
        aggs.append(_sc_gat(_table(h_all[i], as_all[i], ad_all[i]),
                            src3, dst3))
    agg = jnp.stack(aggs)

    h2, as2, ad2 = _layer2(agg, W_out, a_out)
    out = _sc_gat(_table(h2, as2, ad2), src3, dst3)
    return out
